# baseline (speedup 1.0000x reference)
import functools

import jax
import jax.numpy as jnp
from jax import lax
from jax.experimental import pallas as pl
from jax.experimental.pallas import tpu as pltpu
from jax.experimental.pallas import tpu_sc as plsc

_B = 8
_KV_LEN = 8192
_D = 1024
_NC = 2
_NS = 16
_NW = _NC * _NS
_CHUNK = (_B * _D) // _NW
_CPB = _D // _CHUNK

_mesh = plsc.VectorSubcoreMesh(core_axis_name="c", subcore_axis_name="s")


@functools.partial(
    pl.kernel,
    mesh=_mesh,
    out_type=jax.ShapeDtypeStruct((_B, 1, _D), jnp.float32),
)
def _write_narrowed(token_hbm, pos_hbm, kv_hbm, out_hbm):
    del pos_hbm, kv_hbm
    wid = lax.axis_index("s") * _NC + lax.axis_index("c")
    b = wid // _CPB
    col = (wid % _CPB) * _CHUNK
    pltpu.sync_copy(
        token_hbm.at[b, 0, pl.ds(col, _CHUNK)],
        out_hbm.at[b, 0, pl.ds(col, _CHUNK)],
    )


def kernel(input_token, input_pos, kv_cache):
    return _write_narrowed(input_token, input_pos, kv_cache)


# speedup vs baseline: 1.0917x; 1.0917x over previous
import functools

import jax
import jax.numpy as jnp
from jax import lax
from jax.experimental import pallas as pl
from jax.experimental.pallas import tpu as pltpu
from jax.experimental.pallas import tpu_sc as plsc

_B = 8
_KV_LEN = 8192
_D = 1024
_NC = 2
_NS = 16
_NW = _NC * _NS
_CHUNK = (_B * _D) // _NW
_CPB = _D // _CHUNK

_mesh = plsc.ScalarSubcoreMesh(axis_name="c", num_cores=_NC)


@functools.partial(
    pl.kernel,
    mesh=_mesh,
    out_type=jax.ShapeDtypeStruct((_B, 1, _D), jnp.float32),
)
def _write_narrowed(token_hbm, pos_hbm, kv_hbm, out_hbm):
    del pos_hbm, kv_hbm
    c = lax.axis_index("c")
    half = _B // _NC
    pltpu.sync_copy(
        token_hbm.at[pl.ds(c * half, half)],
        out_hbm.at[pl.ds(c * half, half)],
    )


def kernel(input_token, input_pos, kv_cache):
    return _write_narrowed(input_token, input_pos, kv_cache)


# speedup vs baseline: 1.1583x; 1.0610x over previous
import functools

import jax
import jax.numpy as jnp
from jax import lax
from jax.experimental import pallas as pl
from jax.experimental.pallas import tpu as pltpu
from jax.experimental.pallas import tpu_sc as plsc

_B = 8
_KV_LEN = 8192
_D = 1024
_NC = 2
_NS = 16
_NW = _NC * _NS
_CHUNK = (_B * _D) // _NW
_CPB = _D // _CHUNK

_mesh = plsc.ScalarSubcoreMesh(axis_name="c", num_cores=1)


@functools.partial(
    pl.kernel,
    mesh=_mesh,
    out_type=jax.ShapeDtypeStruct((_B, 1, _D), jnp.float32),
)
def _write_narrowed(token_hbm, pos_hbm, kv_hbm, out_hbm):
    del pos_hbm, kv_hbm
    pltpu.sync_copy(token_hbm, out_hbm)


def kernel(input_token, input_pos, kv_cache):
    return _write_narrowed(input_token, input_pos, kv_cache)
